# baseline (speedup 1.0000x reference)
import functools

import jax
import jax.numpy as jnp
from jax import lax
from jax.experimental import pallas as pl
from jax.experimental.pallas import tpu as pltpu
from jax.experimental.pallas import tpu_sc as plsc

N = 10000
M = 20000
NNZ = 320000
D = 128
HF = 64

NC = 2
NS = 16
CH = 128
NCH = 160
SLAB = NCH * CH
PAD_SPREAD = 96

N_PAD = 10240
M_PAD = 20480

_f32 = jnp.float32
_i32 = jnp.int32


def _pad_idx(idx, fill_base):
    per = NNZ // NS
    pad_n = SLAB - per
    idx2 = idx.reshape(NS, per)
    fills = fill_base + (jnp.arange(pad_n, dtype=_i32) % PAD_SPREAD)
    fills2 = jnp.broadcast_to(fills, (NS, pad_n))
    return jnp.concatenate([idx2, fills2], axis=1).reshape(NS, NCH, CH)


_WB = 64
_NE = 8
_NCE = NCH // _NE
_PD = 4


def _make_seg(acc_r, src_r, mode):
    is_v2e = mode.startswith("v2e")
    first = mode.endswith("1")
    r16 = acc_r // NS
    nzb = r16 // _WB

    out_type = [jax.ShapeDtypeStruct((2 * acc_r, HF), _f32)]
    if mode == "v2e1":
        out_type.append(jax.ShapeDtypeStruct((acc_r,), _f32))
    elif mode == "e2v1":
        out_type.append(jax.ShapeDtypeStruct((acc_r,), _f32))
        out_type.append(jax.ShapeDtypeStruct((acc_r,), _f32))

    scratch = [
        pltpu.VMEM((_NCE, CH), _i32),
        pltpu.VMEM((_NCE, CH), _i32),
        pltpu.VMEM((CH, HF), _f32),
        pltpu.VMEM((CH, HF), _f32),
        pltpu.VMEM((CH, HF), _f32),
        pltpu.VMEM((CH, HF), _f32),
        pltpu.VMEM((_WB, HF), _f32),
        pltpu.VMEM((r16,), _f32),
    ]
    if mode == "e2v1":
        scratch.append(pltpu.VMEM((r16,), _f32))
    if first:
        scratch.append(pltpu.VMEM((CH,), _f32))
        scratch.append(pltpu.VMEM_SHARED((acc_r,), _f32))
    scratch.append(pltpu.VMEM_SHARED((acc_r, HF), _f32))
    scratch += [pltpu.SemaphoreType.DMA] * (2 * _PD)

    def seg(*args):
        ins = 4 if mode == "v2e2" else 3
        nout = len(out_type)
        src, gidx, sidx = args[0], args[1], args[2]
        recipe_in = args[3] if mode == "v2e2" else None
        out = args[ins]
        deg_outs = args[ins + 1:ins + nout]
        sc = list(args[ins + nout:])
        gbuf, sbuf, b0, b1, b2, b3, wbuf, rbuf = sc[:8]
        sc = sc[8:]
        mbuf = sc.pop(0) if mode == "e2v1" else None
        if first:
            ones = sc.pop(0)
            deg_sh = sc.pop(0)
        acc_sh = sc.pop(0)
        gsem = sc[:_PD]
        ssem = sc[_PD:]
        bufs = (b0, b1, b2, b3)
        c = lax.axis_index("c")
        s = lax.axis_index("s")
        row0 = s * r16

        @pl.loop(0, _WB)
        def _zw(i):
            for j in range(HF // 16):
                wbuf[i, pl.ds(j * 16, 16)] = jnp.zeros((16,), _f32)

        @pl.loop(0, nzb)
        def _za(r):
            pltpu.sync_copy(wbuf, acc_sh.at[pl.ds(row0 + r * _WB, _WB)])

        if first:
            @pl.loop(0, CH // 16)
            def _fo(i):
                ones[pl.ds(i * 16, 16)] = jnp.ones((16,), _f32)

            @pl.loop(0, r16 // 16)
            def _zr(i):
                rbuf[pl.ds(i * 16, 16)] = jnp.zeros((16,), _f32)

            pltpu.sync_copy(rbuf, deg_sh.at[pl.ds(row0, r16)])

        plsc.subcore_barrier()

        def sg(j, u):
            pltpu.async_copy(src.at[gbuf.at[j]], bufs[u], gsem[u])

        def wg(u):
            pltpu.make_async_copy(src.at[gbuf.at[0]], bufs[u], gsem[u]).wait()

        def ss(j, u):
            pltpu.async_copy(bufs[u], acc_sh.at[sbuf.at[j]], ssem[u], add=True)

        def ws(u):
            pltpu.make_async_copy(
                bufs[u], acc_sh.at[sbuf.at[0]], ssem[u]).wait()

        for hh in range(_NE):
            pltpu.sync_copy(gidx.at[c, s, pl.ds(hh * _NCE, _NCE)], gbuf)
            pltpu.sync_copy(sidx.at[s, pl.ds(hh * _NCE, _NCE)], sbuf)
            for u in range(_PD):
                sg(u, u)

            @pl.loop(0, _NCE // _PD)
            def _main(t):
                j0 = t * _PD
                for u in range(_PD):
                    wg(u)
                    ss(j0 + u, u)
                    if first:
                        pltpu.sync_copy(
                            ones, deg_sh.at[sbuf.at[j0 + u]], add=True)
                for u in range(_PD):
                    @pl.when(j0 + _PD + u < _NCE)
                    def _():
                        ws(u)
                        sg(j0 + _PD + u, u)

            for u in range(_PD):
                ws(u)

        plsc.subcore_barrier()

        if mode == "v2e1":
            pltpu.sync_copy(deg_sh.at[pl.ds(row0, r16)], rbuf)

            @pl.loop(0, r16 // 16)
            def _re(i):
                v = rbuf[pl.ds(i * 16, 16)]
                rbuf[pl.ds(i * 16, 16)] = 1.0 / jnp.maximum(v, 1.0)

            @pl.when(c == 0)
            def _():
                pltpu.sync_copy(rbuf, deg_outs[0].at[pl.ds(row0, r16)])
        elif mode == "v2e2":
            pltpu.sync_copy(recipe_in.at[pl.ds(row0, r16)], rbuf)
        elif mode == "e2v1":
            pltpu.sync_copy(deg_sh.at[pl.ds(row0, r16)], rbuf)

            @pl.loop(0, r16 // 16)
            def _rv(i):
                sl = pl.ds(i * 16, 16)
                v = rbuf[sl]
                rbuf[sl] = 1.0 / jnp.maximum(v, 1.0)
                mbuf[sl] = jnp.where(v > 0.0, 1.0, 0.0).astype(_f32)

            @pl.when(c == 0)
            def _():
                pltpu.sync_copy(rbuf, deg_outs[0].at[pl.ds(row0, r16)])

            @pl.when(c == 1)
            def _():
                pltpu.sync_copy(mbuf, deg_outs[1].at[pl.ds(row0, r16)])

        @pl.loop(0, nzb)
        def _wo(r):
            rows = row0 + r * _WB
            pltpu.sync_copy(acc_sh.at[pl.ds(rows, _WB)], wbuf)
            if is_v2e:
                @pl.loop(0, _WB // 16)
                def _g(g):
                    rvec = rbuf[pl.ds(r * _WB + g * 16, 16)]
                    rows16 = jnp.arange(16, dtype=_i32) + g * 16
                    for kk in range(HF):
                        cols = jnp.full((16,), kk, _i32)
                        vals = plsc.load_gather(wbuf, [rows16, cols])
                        plsc.store_scatter(wbuf, [rows16, cols], vals * rvec)
            pltpu.sync_copy(wbuf, out.at[pl.ds(c * acc_r + rows, _WB)])

    return functools.partial(
        pl.kernel,
        out_type=out_type if len(out_type) > 1 else out_type[0],
        mesh=plsc.VectorSubcoreMesh(core_axis_name="c", subcore_axis_name="s"),
        compiler_params=pltpu.CompilerParams(
            use_tc_tiling_on_sc=False, needs_layout_passes=False),
        scratch_types=scratch,
    )(seg)


_seg_v2e1 = _make_seg(M_PAD, N_PAD, "v2e1")
_seg_v2e2 = _make_seg(M_PAD, N_PAD, "v2e2")
_seg_e2v1 = _make_seg(N_PAD, M_PAD, "e2v1")
_seg_e2v2 = _make_seg(N_PAD, M_PAD, "e2v2")


_BR = 512
_NB = N_PAD // _BR


def _fused_body(a0_ref, a1_ref, r_ref, m_ref, w_ref, b_ref, o_ref):
    y = jnp.concatenate([a0_ref[0], a1_ref[0]], axis=1) * r_ref[...]
    h = jnp.dot(y, w_ref[...], preferred_element_type=_f32)
    res = jnp.maximum(h + m_ref[...] * b_ref[0, :], 0.0)
    o_ref[0] = res[:, :HF]
    o_ref[1] = res[:, HF:]


_fused = pl.pallas_call(
    _fused_body,
    grid=(_NB,),
    in_specs=[
        pl.BlockSpec((1, _BR, HF), lambda i: (0, i, 0)),
        pl.BlockSpec((1, _BR, HF), lambda i: (1, i, 0)),
        pl.BlockSpec((_BR, 1), lambda i: (i, 0)),
        pl.BlockSpec((_BR, 1), lambda i: (i, 0)),
        pl.BlockSpec((D, D), lambda i: (0, 0)),
        pl.BlockSpec((8, D), lambda i: (0, 0)),
    ],
    out_specs=pl.BlockSpec((2, _BR, HF), lambda i: (0, i, 0)),
    out_shape=jax.ShapeDtypeStruct((2, N_PAD, HF), _f32),
)

_BL = 400


def _fused_last_body(a0_ref, a1_ref, r_ref, m_ref, w_ref, b_ref, o_ref):
    y = jnp.concatenate([a0_ref[0], a1_ref[0]], axis=1) * r_ref[...]
    h = jnp.dot(y, w_ref[...], preferred_element_type=_f32)
    o_ref[...] = jnp.maximum(h + m_ref[...] * b_ref[0, :], 0.0)


_fused_last = pl.pallas_call(
    _fused_last_body,
    grid=(N // _BL,),
    in_specs=[
        pl.BlockSpec((1, _BL, HF), lambda i: (0, i, 0)),
        pl.BlockSpec((1, _BL, HF), lambda i: (1, i, 0)),
        pl.BlockSpec((_BL, 1), lambda i: (i, 0)),
        pl.BlockSpec((_BL, 1), lambda i: (i, 0)),
        pl.BlockSpec((D, D), lambda i: (0, 0)),
        pl.BlockSpec((8, D), lambda i: (0, 0)),
    ],
    out_specs=pl.BlockSpec((_BL, D), lambda i: (i, 0)),
    out_shape=jax.ShapeDtypeStruct((N, D), _f32),
)


@jax.jit
def kernel(x, v_idx, e_idx, W0, b0, W1, b1):
    v_idx = v_idx.astype(_i32)
    e_idx = e_idx.astype(_i32)

    vpad = _pad_idx(v_idx, N)
    epad = _pad_idx(e_idx, M)
    vg = jnp.stack([vpad, vpad + N_PAD])
    eg = jnp.stack([epad, epad + M_PAD])

    xp = jnp.zeros((N_PAD, D), _f32).at[:N].set(x)
    xs = jnp.concatenate([xp[:, :HF], xp[:, HF:]], axis=0)

    b0b = jnp.broadcast_to(b0, (8, D))
    b1b = jnp.broadcast_to(b1, (8, D))

    e_s, recip_e = _seg_v2e1(xs, vg, epad)
    v_acc, recip_v, mask_v = _seg_e2v1(e_s, eg, vpad)
    rv = recip_v.reshape(N_PAD, 1)
    mv = mask_v.reshape(N_PAD, 1)
    va3 = v_acc.reshape(2, N_PAD, HF)
    xs = _fused(va3, va3, rv, mv, W0, b0b).reshape(2 * N_PAD, HF)

    e_s = _seg_v2e2(xs, vg, epad, recip_e)
    v_acc = _seg_e2v2(e_s, eg, vpad)
    va3 = v_acc.reshape(2, N_PAD, HF)
    return _fused_last(va3, va3, rv, mv, W1, b1b)


# speedup vs baseline: 1.2600x; 1.2600x over previous
import functools

import jax
import jax.numpy as jnp
from jax import lax
from jax.experimental import pallas as pl
from jax.experimental.pallas import tpu as pltpu
from jax.experimental.pallas import tpu_sc as plsc

N = 10000
M = 20000
NNZ = 320000
D = 128
HF = 64

NC = 2
NS = 16
CH = 128
NCH = 160
SLAB = NCH * CH
PAD_SPREAD = 96

N_PAD = 10240
M_PAD = 20480

_f32 = jnp.float32
_i32 = jnp.int32


def _pad_idx(idx, fill_base):
    per = NNZ // NS
    pad_n = SLAB - per
    idx2 = idx.reshape(NS, per)
    fills = fill_base + (jnp.arange(pad_n, dtype=_i32) % PAD_SPREAD)
    fills2 = jnp.broadcast_to(fills, (NS, pad_n))
    return jnp.concatenate([idx2, fills2], axis=1).reshape(NS, NCH, CH)


_ME16 = M_PAD // NS
_NV16 = N_PAD // NS


@functools.partial(
    pl.kernel,
    out_type=[
        jax.ShapeDtypeStruct((M_PAD,), _f32),
        jax.ShapeDtypeStruct((N_PAD,), _f32),
        jax.ShapeDtypeStruct((N_PAD,), _f32),
    ],
    mesh=plsc.VectorSubcoreMesh(core_axis_name="c", subcore_axis_name="s"),
    compiler_params=pltpu.CompilerParams(use_tc_tiling_on_sc=False),
    scratch_types=[
        pltpu.VMEM((NCH, CH), _i32),
        pltpu.VMEM((NCH, CH), _i32),
        pltpu.VMEM((CH,), _f32),
        pltpu.VMEM((_ME16,), _f32),
        pltpu.VMEM((_ME16,), _f32),
        pltpu.VMEM_SHARED((M_PAD,), _f32),
        pltpu.VMEM_SHARED((N_PAD,), _f32),
    ],
)
def _deg_kernel(vslab, eslab, recip_e, recip_v, mask_v,
                vbuf, ebuf, ones, sbuf, obuf, de_sh, dv_sh):
    c = lax.axis_index("c")
    s = lax.axis_index("s")

    @pl.loop(0, CH // 16)
    def _fill(i):
        ones[pl.ds(i * 16, 16)] = jnp.ones((16,), _f32)
        sbuf[pl.ds(i * 16, 16)] = jnp.zeros((16,), _f32)

    @pl.loop(CH // 16, _ME16 // 16)
    def _z(i):
        sbuf[pl.ds(i * 16, 16)] = jnp.zeros((16,), _f32)

    pltpu.sync_copy(sbuf, de_sh.at[pl.ds(s * _ME16, _ME16)])
    pltpu.sync_copy(sbuf.at[pl.ds(0, _NV16)], dv_sh.at[pl.ds(s * _NV16, _NV16)])
    plsc.subcore_barrier()

    pltpu.sync_copy(vslab.at[s], vbuf)
    pltpu.sync_copy(eslab.at[s], ebuf)

    @pl.loop(0, NCH)
    def _acc(k):
        pltpu.sync_copy(ones, dv_sh.at[vbuf.at[k]], add=True)
        pltpu.sync_copy(ones, de_sh.at[ebuf.at[k]], add=True)

    plsc.subcore_barrier()

    pltpu.sync_copy(de_sh.at[pl.ds(s * _ME16, _ME16)], sbuf)

    @pl.loop(0, _ME16 // 16)
    def _re(i):
        v = sbuf[pl.ds(i * 16, 16)]
        obuf[pl.ds(i * 16, 16)] = 1.0 / jnp.maximum(v, 1.0)

    @pl.when(c == 0)
    def _():
        pltpu.sync_copy(obuf, recip_e.at[pl.ds(s * _ME16, _ME16)])

    pltpu.sync_copy(dv_sh.at[pl.ds(s * _NV16, _NV16)], sbuf.at[pl.ds(0, _NV16)])

    @pl.loop(0, _NV16 // 16)
    def _rv(i):
        v = sbuf[pl.ds(i * 16, 16)]
        obuf[pl.ds(i * 16, 16)] = 1.0 / jnp.maximum(v, 1.0)
        obuf[pl.ds(_NV16 + i * 16, 16)] = jnp.where(
            v > 0.0, jnp.ones((16,), _f32), jnp.zeros((16,), _f32))

    @pl.when(c == 1)
    def _():
        pltpu.sync_copy(obuf.at[pl.ds(0, _NV16)], recip_v.at[pl.ds(s * _NV16, _NV16)])
        pltpu.sync_copy(obuf.at[pl.ds(_NV16, _NV16)], mask_v.at[pl.ds(s * _NV16, _NV16)])


_WB = 64
_NE = 8
_NCE = NCH // _NE
_PD = 4


def _make_seg(acc_r, src_r, scale):
    r16 = acc_r // NS
    nzb = r16 // _WB

    scratch = [
        pltpu.VMEM((_NCE, CH), _i32),
        pltpu.VMEM((_NCE, CH), _i32),
        pltpu.VMEM((CH, HF), _f32),
        pltpu.VMEM((CH, HF), _f32),
        pltpu.VMEM((CH, HF), _f32),
        pltpu.VMEM((CH, HF), _f32),
        pltpu.VMEM((_WB, HF), _f32),
        pltpu.VMEM((_WB, HF), _f32),
        pltpu.VMEM_SHARED((acc_r, HF), _f32),
    ] + [pltpu.SemaphoreType.DMA] * (2 * _PD)

    def seg(src, gidx, sidx, rb_hbm, out, gbuf, sbuf, b0, b1, b2, b3,
            wbuf, rbw, acc_sh, *sems):
        gsem = sems[:_PD]
        ssem = sems[_PD:]
        bufs = (b0, b1, b2, b3)
        c = lax.axis_index("c")
        s = lax.axis_index("s")
        row0 = s * r16

        @pl.loop(0, _WB)
        def _zw(i):
            for j in range(HF // 16):
                wbuf[i, pl.ds(j * 16, 16)] = jnp.zeros((16,), _f32)

        @pl.loop(0, nzb)
        def _za(r):
            pltpu.sync_copy(wbuf, acc_sh.at[pl.ds(row0 + r * _WB, _WB)])

        plsc.subcore_barrier()

        def sg(j, u):
            pltpu.async_copy(src.at[gbuf.at[j]], bufs[u], gsem[u])

        def wg(u):
            pltpu.make_async_copy(src.at[gbuf.at[0]], bufs[u], gsem[u]).wait()

        def ss(j, u):
            pltpu.async_copy(bufs[u], acc_sh.at[sbuf.at[j]], ssem[u], add=True)

        def ws(u):
            pltpu.make_async_copy(
                bufs[u], acc_sh.at[sbuf.at[0]], ssem[u]).wait()

        for hh in range(_NE):
            pltpu.sync_copy(gidx.at[c, s, pl.ds(hh * _NCE, _NCE)], gbuf)
            pltpu.sync_copy(sidx.at[s, pl.ds(hh * _NCE, _NCE)], sbuf)
            for u in range(_PD):
                sg(u, u)

            @pl.loop(0, _NCE // _PD)
            def _main(t):
                j0 = t * _PD
                for u in range(_PD):
                    wg(u)
                    ss(j0 + u, u)
                for u in range(_PD):
                    @pl.when(j0 + _PD + u < _NCE)
                    def _():
                        ws(u)
                        sg(j0 + _PD + u, u)

            for u in range(_PD):
                ws(u)

        plsc.subcore_barrier()

        @pl.loop(0, nzb)
        def _wo(r):
            rows = row0 + r * _WB
            pltpu.sync_copy(acc_sh.at[pl.ds(rows, _WB)], wbuf)
            if scale:
                pltpu.sync_copy(rb_hbm.at[pl.ds(rows, _WB)], rbw)

                @pl.loop(0, _WB)
                def _m(i):
                    for j in range(HF // 16):
                        sl = pl.ds(j * 16, 16)
                        wbuf[i, sl] = wbuf[i, sl] * rbw[i, sl]
            pltpu.sync_copy(wbuf, out.at[pl.ds(c * acc_r + rows, _WB)])

    return functools.partial(
        pl.kernel,
        out_type=jax.ShapeDtypeStruct((2 * acc_r, HF), _f32),
        mesh=plsc.VectorSubcoreMesh(core_axis_name="c", subcore_axis_name="s"),
        compiler_params=pltpu.CompilerParams(use_tc_tiling_on_sc=False),
        scratch_types=scratch,
    )(seg)


_seg_v2e = _make_seg(M_PAD, N_PAD, True)
_seg_e2v = _make_seg(N_PAD, M_PAD, False)


_BR = 512
_NB = N_PAD // _BR


def _fused_body(a0_ref, a1_ref, r_ref, m_ref, w_ref, b_ref, o_ref):
    y = jnp.concatenate([a0_ref[0], a1_ref[0]], axis=1) * r_ref[...]
    h = jnp.dot(y, w_ref[...], preferred_element_type=_f32)
    res = jnp.maximum(h + m_ref[...] * b_ref[0, :], 0.0)
    o_ref[0] = res[:, :HF]
    o_ref[1] = res[:, HF:]


_fused = pl.pallas_call(
    _fused_body,
    grid=(_NB,),
    in_specs=[
        pl.BlockSpec((1, _BR, HF), lambda i: (0, i, 0)),
        pl.BlockSpec((1, _BR, HF), lambda i: (1, i, 0)),
        pl.BlockSpec((_BR, 1), lambda i: (i, 0)),
        pl.BlockSpec((_BR, 1), lambda i: (i, 0)),
        pl.BlockSpec((D, D), lambda i: (0, 0)),
        pl.BlockSpec((8, D), lambda i: (0, 0)),
    ],
    out_specs=pl.BlockSpec((2, _BR, HF), lambda i: (0, i, 0)),
    out_shape=jax.ShapeDtypeStruct((2, N_PAD, HF), _f32),
)

_BL = 400


def _fused_last_body(a0_ref, a1_ref, r_ref, m_ref, w_ref, b_ref, o_ref):
    y = jnp.concatenate([a0_ref[0], a1_ref[0]], axis=1) * r_ref[...]
    h = jnp.dot(y, w_ref[...], preferred_element_type=_f32)
    o_ref[...] = jnp.maximum(h + m_ref[...] * b_ref[0, :], 0.0)


_fused_last = pl.pallas_call(
    _fused_last_body,
    grid=(N // _BL,),
    in_specs=[
        pl.BlockSpec((1, _BL, HF), lambda i: (0, i, 0)),
        pl.BlockSpec((1, _BL, HF), lambda i: (1, i, 0)),
        pl.BlockSpec((_BL, 1), lambda i: (i, 0)),
        pl.BlockSpec((_BL, 1), lambda i: (i, 0)),
        pl.BlockSpec((D, D), lambda i: (0, 0)),
        pl.BlockSpec((8, D), lambda i: (0, 0)),
    ],
    out_specs=pl.BlockSpec((_BL, D), lambda i: (i, 0)),
    out_shape=jax.ShapeDtypeStruct((N, D), _f32),
)


@jax.jit
def kernel(x, v_idx, e_idx, W0, b0, W1, b1):
    v_idx = v_idx.astype(_i32)
    e_idx = e_idx.astype(_i32)

    vpad = _pad_idx(v_idx, N)
    epad = _pad_idx(e_idx, M)
    vg = jnp.stack([vpad, vpad + N_PAD])
    eg = jnp.stack([epad, epad + M_PAD])

    recip_e, recip_v, mask_v = _deg_kernel(vpad, epad)
    rbe = jnp.broadcast_to(recip_e[:, None], (M_PAD, HF)) + jnp.zeros(
        (M_PAD, HF), _f32)
    rv = recip_v.reshape(N_PAD, 1)
    mv = mask_v.reshape(N_PAD, 1)

    xp = jnp.zeros((N_PAD, D), _f32).at[:N].set(x)
    xs = jnp.concatenate([xp[:, :HF], xp[:, HF:]], axis=0)

    b0b = jnp.broadcast_to(b0, (8, D))
    b1b = jnp.broadcast_to(b1, (8, D))

    e_s = _seg_v2e(xs, vg, epad, rbe)
    v_acc = _seg_e2v(e_s, eg, vpad, rbe)
    va3 = v_acc.reshape(2, N_PAD, HF)
    xs = _fused(va3, va3, rv, mv, W0, b0b).reshape(2 * N_PAD, HF)

    e_s = _seg_v2e(xs, vg, epad, rbe)
    v_acc = _seg_e2v(e_s, eg, vpad, rbe)
    va3 = v_acc.reshape(2, N_PAD, HF)
    return _fused_last(va3, va3, rv, mv, W1, b1b)


# speedup vs baseline: 1.3404x; 1.0638x over previous
import functools

import jax
import jax.numpy as jnp
from jax import lax
from jax.experimental import pallas as pl
from jax.experimental.pallas import tpu as pltpu
from jax.experimental.pallas import tpu_sc as plsc

N = 10000
M = 20000
NNZ = 320000
D = 128
HF = 64

NC = 2
NS = 16
CH = 128
NCH = 160
SLAB = NCH * CH
PAD_SPREAD = 96

N_PAD = 10240
M_PAD = 20480

_f32 = jnp.float32
_i32 = jnp.int32


def _pad_idx(idx, fill_base):
    per = NNZ // NS
    pad_n = SLAB - per
    idx2 = idx.reshape(NS, per)
    fills = fill_base + (jnp.arange(pad_n, dtype=_i32) % PAD_SPREAD)
    fills2 = jnp.broadcast_to(fills, (NS, pad_n))
    return jnp.concatenate([idx2, fills2], axis=1).reshape(NS, NCH, CH)


_ME16 = M_PAD // NS
_NV16 = N_PAD // NS


@functools.partial(
    pl.kernel,
    out_type=[
        jax.ShapeDtypeStruct((M_PAD,), _f32),
        jax.ShapeDtypeStruct((N_PAD,), _f32),
        jax.ShapeDtypeStruct((N_PAD,), _f32),
    ],
    mesh=plsc.VectorSubcoreMesh(core_axis_name="c", subcore_axis_name="s"),
    compiler_params=pltpu.CompilerParams(use_tc_tiling_on_sc=False),
    scratch_types=[
        pltpu.VMEM((NCH, CH), _i32),
        pltpu.VMEM((NCH, CH), _i32),
        pltpu.VMEM((CH,), _f32),
        pltpu.VMEM((_ME16,), _f32),
        pltpu.VMEM((_ME16,), _f32),
        pltpu.VMEM_SHARED((M_PAD,), _f32),
        pltpu.VMEM_SHARED((N_PAD,), _f32),
        pltpu.SemaphoreType.DMA,
        pltpu.SemaphoreType.DMA,
    ],
)
def _deg_kernel(vslab, eslab, recip_e, recip_v, mask_v,
                vbuf, ebuf, ones, sbuf, obuf, de_sh, dv_sh, vsem, esem):
    c = lax.axis_index("c")
    s = lax.axis_index("s")

    @pl.loop(0, CH // 16)
    def _fill(i):
        ones[pl.ds(i * 16, 16)] = jnp.ones((16,), _f32)
        sbuf[pl.ds(i * 16, 16)] = jnp.zeros((16,), _f32)

    @pl.loop(CH // 16, _ME16 // 16)
    def _z(i):
        sbuf[pl.ds(i * 16, 16)] = jnp.zeros((16,), _f32)

    pltpu.sync_copy(sbuf, de_sh.at[pl.ds(s * _ME16, _ME16)])
    pltpu.sync_copy(sbuf.at[pl.ds(0, _NV16)], dv_sh.at[pl.ds(s * _NV16, _NV16)])
    plsc.subcore_barrier()

    pltpu.sync_copy(vslab.at[s], vbuf)
    pltpu.sync_copy(eslab.at[s], ebuf)

    @pl.loop(0, NCH)
    def _acc(k):
        pltpu.async_copy(ones, dv_sh.at[vbuf.at[k]], vsem, add=True)
        pltpu.async_copy(ones, de_sh.at[ebuf.at[k]], esem, add=True)

    @pl.loop(0, NCH)
    def _drain(k):
        pltpu.make_async_copy(ones, dv_sh.at[vbuf.at[0]], vsem).wait()
        pltpu.make_async_copy(ones, de_sh.at[ebuf.at[0]], esem).wait()

    plsc.subcore_barrier()

    pltpu.sync_copy(de_sh.at[pl.ds(s * _ME16, _ME16)], sbuf)

    @pl.loop(0, _ME16 // 16)
    def _re(i):
        v = sbuf[pl.ds(i * 16, 16)]
        obuf[pl.ds(i * 16, 16)] = 1.0 / jnp.maximum(v, 1.0)

    @pl.when(c == 0)
    def _():
        pltpu.sync_copy(obuf, recip_e.at[pl.ds(s * _ME16, _ME16)])

    pltpu.sync_copy(dv_sh.at[pl.ds(s * _NV16, _NV16)], sbuf.at[pl.ds(0, _NV16)])

    @pl.loop(0, _NV16 // 16)
    def _rv(i):
        v = sbuf[pl.ds(i * 16, 16)]
        obuf[pl.ds(i * 16, 16)] = 1.0 / jnp.maximum(v, 1.0)
        obuf[pl.ds(_NV16 + i * 16, 16)] = jnp.where(
            v > 0.0, jnp.ones((16,), _f32), jnp.zeros((16,), _f32))

    @pl.when(c == 1)
    def _():
        pltpu.sync_copy(obuf.at[pl.ds(0, _NV16)], recip_v.at[pl.ds(s * _NV16, _NV16)])
        pltpu.sync_copy(obuf.at[pl.ds(_NV16, _NV16)], mask_v.at[pl.ds(s * _NV16, _NV16)])


_WB = 64
_NE = 5
_NCE = NCH // _NE
_PD = 4


def _make_seg(acc_r, src_r, scale):
    r16 = acc_r // NS
    nzb = r16 // _WB

    scratch = [
        pltpu.VMEM((_NCE, CH), _i32),
        pltpu.VMEM((_NCE, CH), _i32),
        pltpu.VMEM((CH, HF), _f32),
        pltpu.VMEM((CH, HF), _f32),
        pltpu.VMEM((CH, HF), _f32),
        pltpu.VMEM((CH, HF), _f32),
        pltpu.VMEM((_WB, HF), _f32),
        pltpu.VMEM((_WB, HF), _f32),
        pltpu.VMEM_SHARED((acc_r, HF), _f32),
    ] + [pltpu.SemaphoreType.DMA] * (2 * _PD)

    def seg(src, gidx, sidx, rb_hbm, out, gbuf, sbuf, b0, b1, b2, b3,
            wbuf, rbw, acc_sh, *sems):
        gsem = sems[:_PD]
        ssem = sems[_PD:]
        bufs = (b0, b1, b2, b3)
        c = lax.axis_index("c")
        s = lax.axis_index("s")
        row0 = s * r16

        @pl.loop(0, _WB)
        def _zw(i):
            for j in range(HF // 16):
                wbuf[i, pl.ds(j * 16, 16)] = jnp.zeros((16,), _f32)

        @pl.loop(0, nzb)
        def _za(r):
            pltpu.sync_copy(wbuf, acc_sh.at[pl.ds(row0 + r * _WB, _WB)])

        plsc.subcore_barrier()

        def sg(j, u):
            pltpu.async_copy(src.at[gbuf.at[j]], bufs[u], gsem[u])

        def wg(u):
            pltpu.make_async_copy(src.at[gbuf.at[0]], bufs[u], gsem[u]).wait()

        def ss(j, u):
            pltpu.async_copy(bufs[u], acc_sh.at[sbuf.at[j]], ssem[u], add=True)

        def ws(u):
            pltpu.make_async_copy(
                bufs[u], acc_sh.at[sbuf.at[0]], ssem[u]).wait()

        for hh in range(_NE):
            pltpu.sync_copy(gidx.at[c, s, pl.ds(hh * _NCE, _NCE)], gbuf)
            pltpu.sync_copy(sidx.at[s, pl.ds(hh * _NCE, _NCE)], sbuf)
            for u in range(_PD):
                sg(u, u)

            @pl.loop(0, _NCE // _PD)
            def _main(t):
                j0 = t * _PD
                for u in range(_PD):
                    wg(u)
                    ss(j0 + u, u)
                for u in range(_PD):
                    @pl.when(j0 + _PD + u < _NCE)
                    def _():
                        ws(u)
                        sg(j0 + _PD + u, u)

            for u in range(_PD):
                ws(u)

        plsc.subcore_barrier()

        @pl.loop(0, nzb)
        def _wo(r):
            rows = row0 + r * _WB
            pltpu.sync_copy(acc_sh.at[pl.ds(rows, _WB)], wbuf)
            if scale:
                pltpu.sync_copy(rb_hbm.at[pl.ds(rows, _WB)], rbw)

                @pl.loop(0, _WB)
                def _m(i):
                    for j in range(HF // 16):
                        sl = pl.ds(j * 16, 16)
                        wbuf[i, sl] = wbuf[i, sl] * rbw[i, sl]
            pltpu.sync_copy(wbuf, out.at[pl.ds(c * acc_r + rows, _WB)])

    return functools.partial(
        pl.kernel,
        out_type=jax.ShapeDtypeStruct((2 * acc_r, HF), _f32),
        mesh=plsc.VectorSubcoreMesh(core_axis_name="c", subcore_axis_name="s"),
        compiler_params=pltpu.CompilerParams(use_tc_tiling_on_sc=False),
        scratch_types=scratch,
    )(seg)


_seg_v2e = _make_seg(M_PAD, N_PAD, True)
_seg_e2v = _make_seg(N_PAD, M_PAD, False)


_BR = 512
_NB = N_PAD // _BR


def _fused_body(a0_ref, a1_ref, r_ref, m_ref, w_ref, b_ref, o_ref):
    y = jnp.concatenate([a0_ref[0], a1_ref[0]], axis=1) * r_ref[...]
    h = jnp.dot(y, w_ref[...], preferred_element_type=_f32)
    res = jnp.maximum(h + m_ref[...] * b_ref[0, :], 0.0)
    o_ref[0] = res[:, :HF]
    o_ref[1] = res[:, HF:]


_fused = pl.pallas_call(
    _fused_body,
    grid=(_NB,),
    in_specs=[
        pl.BlockSpec((1, _BR, HF), lambda i: (0, i, 0)),
        pl.BlockSpec((1, _BR, HF), lambda i: (1, i, 0)),
        pl.BlockSpec((_BR, 1), lambda i: (i, 0)),
        pl.BlockSpec((_BR, 1), lambda i: (i, 0)),
        pl.BlockSpec((D, D), lambda i: (0, 0)),
        pl.BlockSpec((8, D), lambda i: (0, 0)),
    ],
    out_specs=pl.BlockSpec((2, _BR, HF), lambda i: (0, i, 0)),
    out_shape=jax.ShapeDtypeStruct((2, N_PAD, HF), _f32),
)

_BL = 400


def _fused_last_body(a0_ref, a1_ref, r_ref, m_ref, w_ref, b_ref, o_ref):
    y = jnp.concatenate([a0_ref[0], a1_ref[0]], axis=1) * r_ref[...]
    h = jnp.dot(y, w_ref[...], preferred_element_type=_f32)
    o_ref[...] = jnp.maximum(h + m_ref[...] * b_ref[0, :], 0.0)


_fused_last = pl.pallas_call(
    _fused_last_body,
    grid=(N // _BL,),
    in_specs=[
        pl.BlockSpec((1, _BL, HF), lambda i: (0, i, 0)),
        pl.BlockSpec((1, _BL, HF), lambda i: (1, i, 0)),
        pl.BlockSpec((_BL, 1), lambda i: (i, 0)),
        pl.BlockSpec((_BL, 1), lambda i: (i, 0)),
        pl.BlockSpec((D, D), lambda i: (0, 0)),
        pl.BlockSpec((8, D), lambda i: (0, 0)),
    ],
    out_specs=pl.BlockSpec((_BL, D), lambda i: (i, 0)),
    out_shape=jax.ShapeDtypeStruct((N, D), _f32),
)


@jax.jit
def kernel(x, v_idx, e_idx, W0, b0, W1, b1):
    v_idx = v_idx.astype(_i32)
    e_idx = e_idx.astype(_i32)

    vpad = _pad_idx(v_idx, N)
    epad = _pad_idx(e_idx, M)
    vg = jnp.stack([vpad, vpad + N_PAD])
    eg = jnp.stack([epad, epad + M_PAD])

    recip_e, recip_v, mask_v = _deg_kernel(vpad, epad)
    rbe = jnp.broadcast_to(recip_e[:, None], (M_PAD, HF)) + jnp.zeros(
        (M_PAD, HF), _f32)
    rv = recip_v.reshape(N_PAD, 1)
    mv = mask_v.reshape(N_PAD, 1)

    xp = jnp.zeros((N_PAD, D), _f32).at[:N].set(x)
    xs = jnp.concatenate([xp[:, :HF], xp[:, HF:]], axis=0)

    b0b = jnp.broadcast_to(b0, (8, D))
    b1b = jnp.broadcast_to(b1, (8, D))

    e_s = _seg_v2e(xs, vg, epad, rbe)
    v_acc = _seg_e2v(e_s, eg, vpad, rbe)
    va3 = v_acc.reshape(2, N_PAD, HF)
    xs = _fused(va3, va3, rv, mv, W0, b0b).reshape(2 * N_PAD, HF)

    e_s = _seg_v2e(xs, vg, epad, rbe)
    v_acc = _seg_e2v(e_s, eg, vpad, rbe)
    va3 = v_acc.reshape(2, N_PAD, HF)
    return _fused_last(va3, va3, rv, mv, W1, b1b)
